# Initial kernel scaffold; baseline (speedup 1.0000x reference)
import jax
import jax.numpy as jnp
from jax.experimental import pallas as pl


def kernel(x, edge_index, W1, a_src1, a_dst1, b1, W2, a_src2, a_dst2, b2):
    raise NotImplementedError("write your pallas kernel here")


# speedup vs baseline: 18.7182x; 18.7182x over previous
import functools

import jax
import jax.numpy as jnp
from jax import lax
from jax.experimental import pallas as pl
from jax.experimental.pallas import tpu as pltpu
from jax.experimental.pallas import tpu_sc as plsc

H1 = 8
HID = 16
OUT = 64

NSUB = 16
NCORE = 2
NW = NSUB * NCORE
CH = 256
CB = CH // 128
GRP = 4

BR = 1000

NEG = -3.0e38


def _tc1_body(x_ref, w_ref, asrc_ref, adst_ref, ha_ref, hb_ref,
              s_ref, d_ref, m_ref):
    i = pl.program_id(0)
    hb = jnp.dot(x_ref[...], w_ref[...], preferred_element_type=jnp.float32)
    ha_ref[...] = hb[:, :OUT]
    hb_ref[...] = hb[:, OUT:]
    h3 = hb.reshape(BR, H1, HID)
    sb = jnp.sum(h3 * asrc_ref[...][None], axis=-1)
    db = jnp.sum(h3 * adst_ref[...][None], axis=-1)
    z = jnp.zeros((BR, 16 - H1), jnp.float32)
    s_ref[...] = jnp.concatenate([sb, z], axis=1)
    d_ref[...] = jnp.concatenate([db, z], axis=1)
    mrow = jnp.concatenate(
        [jnp.full((1, 128), jnp.max(sb)), jnp.full((1, 128), jnp.max(db)),
         jnp.full((6, 128), NEG)], axis=0)

    @pl.when(i == 0)
    def _():
        m_ref[...] = mrow

    @pl.when(i > 0)
    def _():
        m_ref[...] = jnp.maximum(m_ref[...], mrow)


def _layer1_dense(x, W1, a_src1, a_dst1):
    n = x.shape[0]
    grid = n // BR
    return pl.pallas_call(
        _tc1_body,
        grid=(grid,),
        in_specs=[
            pl.BlockSpec((BR, 128), lambda i: (i, 0)),
            pl.BlockSpec((128, 128), lambda i: (0, 0)),
            pl.BlockSpec((H1, HID), lambda i: (0, 0)),
            pl.BlockSpec((H1, HID), lambda i: (0, 0)),
        ],
        out_specs=[
            pl.BlockSpec((BR, OUT), lambda i: (i, 0)),
            pl.BlockSpec((BR, OUT), lambda i: (i, 0)),
            pl.BlockSpec((BR, 16), lambda i: (i, 0)),
            pl.BlockSpec((BR, 16), lambda i: (i, 0)),
            pl.BlockSpec((8, 128), lambda i: (0, 0)),
        ],
        out_shape=[
            jax.ShapeDtypeStruct((n, OUT), jnp.float32),
            jax.ShapeDtypeStruct((n, OUT), jnp.float32),
            jax.ShapeDtypeStruct((n, 16), jnp.float32),
            jax.ShapeDtypeStruct((n, 16), jnp.float32),
            jax.ShapeDtypeStruct((8, 128), jnp.float32),
        ],
    )(x, W1, a_src1, a_dst1)


def _inv_body(d_ref, o_ref):
    o_ref[...] = 1.0 / (d_ref[0] + d_ref[1] + 1e-16)


def _combine_inv(den):
    npad = den.shape[1]
    return pl.pallas_call(
        _inv_body,
        out_shape=jax.ShapeDtypeStruct((npad, 16), jnp.float32),
    )(den)


def _tc2_body(a_ref, b_ref, b1_ref, w2_ref, a2s_ref, a2d_ref,
              h2_ref, s_ref, d_ref, m_ref):
    i = pl.program_id(0)
    o = jnp.concatenate([a_ref[0] + a_ref[1], b_ref[0] + b_ref[1]], axis=1)
    o = o + b1_ref[...]
    o = jnp.where(o > 0, o, jnp.exp(o) - 1.0)
    h2 = jnp.dot(o, w2_ref[...], preferred_element_type=jnp.float32)
    h2_ref[...] = h2
    sb = jnp.dot(h2, a2s_ref[...], preferred_element_type=jnp.float32)
    db = jnp.dot(h2, a2d_ref[...], preferred_element_type=jnp.float32)
    s_ref[...] = sb
    d_ref[...] = db
    mrow = jnp.concatenate(
        [jnp.full((1, 128), jnp.max(sb[:, 0])),
         jnp.full((1, 128), jnp.max(db[:, 0])),
         jnp.full((6, 128), NEG)], axis=0)

    @pl.when(i == 0)
    def _():
        m_ref[...] = mrow

    @pl.when(i > 0)
    def _():
        m_ref[...] = jnp.maximum(m_ref[...], mrow)


def _layer2_dense(acc1a, acc1b, b1r, W2, A2s, A2d, n):
    grid = n // BR
    return pl.pallas_call(
        _tc2_body,
        grid=(grid,),
        in_specs=[
            pl.BlockSpec((2, BR, OUT), lambda i: (0, i, 0)),
            pl.BlockSpec((2, BR, OUT), lambda i: (0, i, 0)),
            pl.BlockSpec((1, 128), lambda i: (0, 0)),
            pl.BlockSpec((128, OUT), lambda i: (0, 0)),
            pl.BlockSpec((OUT, 16), lambda i: (0, 0)),
            pl.BlockSpec((OUT, 16), lambda i: (0, 0)),
        ],
        out_specs=[
            pl.BlockSpec((BR, OUT), lambda i: (i, 0)),
            pl.BlockSpec((BR, 16), lambda i: (i, 0)),
            pl.BlockSpec((BR, 16), lambda i: (i, 0)),
            pl.BlockSpec((8, 128), lambda i: (0, 0)),
        ],
        out_shape=[
            jax.ShapeDtypeStruct((n, OUT), jnp.float32),
            jax.ShapeDtypeStruct((n, 16), jnp.float32),
            jax.ShapeDtypeStruct((n, 16), jnp.float32),
            jax.ShapeDtypeStruct((8, 128), jnp.float32),
        ],
    )(acc1a, acc1b, b1r, W2, A2s, A2d)


def _fin_body(a_ref, b_ref, o_ref):
    o_ref[...] = a_ref[0] + a_ref[1] + b_ref[...]


def _final_sum(acc2, b2r, n):
    grid = n // BR
    return pl.pallas_call(
        _fin_body,
        grid=(grid,),
        in_specs=[
            pl.BlockSpec((2, BR, OUT), lambda i: (0, i, 0)),
            pl.BlockSpec((1, OUT), lambda i: (0, 0)),
        ],
        out_specs=pl.BlockSpec((BR, OUT), lambda i: (i, 0)),
        out_shape=jax.ShapeDtypeStruct((n, OUT), jnp.float32),
    )(acc2, b2r)


def _mesh():
    return plsc.VectorSubcoreMesh(
        core_axis_name="c", subcore_axis_name="s",
        num_cores=NCORE, num_subcores=NSUB)


def _att_phase(src2, dst2, as_t, ad_t, m16, zrows, ep, npad):
    te = ep // NW
    nch = te // CH
    stripe = npad // NSUB
    np16 = as_t.shape[0]
    tstr = np16 // NSUB

    def body(src_h, dst_h, as_h, ad_h, m_h, z_h, e_h, den_h,
             sidx, didx, srow, drow, erow, mv, sem, den_sh):
        c = lax.axis_index("c")
        s = lax.axis_index("s")
        wid = s * NCORE + c
        pltpu.sync_copy(z_h, den_sh.at[pl.ds(s * stripe, stripe)])
        pltpu.sync_copy(m_h, mv)
        plsc.subcore_barrier()
        m = mv[...]

        def group(g, carry):
            rb = wid * (te // 128) + g * (GRP * CB)
            pltpu.sync_copy(src_h.at[pl.ds(rb, GRP * CB)], sidx)
            pltpu.sync_copy(dst_h.at[pl.ds(rb, GRP * CB)], didx)
            for sub in range(GRP):
                eb8 = wid * (te // 8) + (g * GRP + sub) * (CH // 8)
                descs = []
                for jj in range(CB):
                    j = sub * CB + jj
                    descs.append(pltpu.async_copy(
                        as_h.at[sidx.at[j]],
                        srow.at[pl.ds(jj * 128, 128)], sem))
                    descs.append(pltpu.async_copy(
                        ad_h.at[didx.at[j]],
                        drow.at[pl.ds(jj * 128, 128)], sem))
                for dsc in descs:
                    dsc.wait()

                def edge(k, carry2):
                    z = srow[k, :] + drow[k, :]
                    zl = jnp.where(z >= 0.0, z, 0.2 * z) - m
                    ev = jnp.exp(zl)
                    srow[k, :] = ev
                    erow[k // 8, pl.ds((k % 8) * 16, 16)] = ev
                    return carry2

                lax.fori_loop(0, CH, edge, 0)
                pltpu.sync_copy(erow, e_h.at[pl.ds(eb8, CH // 8)])
                for jj in range(CB):
                    j = sub * CB + jj
                    pltpu.sync_copy(
                        srow.at[pl.ds(jj * 128, 128)],
                        den_sh.at[didx.at[j]], add=True)
            return carry

        lax.fori_loop(0, nch // GRP, group, 0)
        plsc.subcore_barrier()
        pltpu.sync_copy(den_sh.at[pl.ds(s * stripe, stripe)],
                        den_h.at[c, pl.ds(s * stripe, stripe)])

    call = pl.kernel(
        body,
        out_type=[
            jax.ShapeDtypeStruct((ep // 8, 128), jnp.float32),
            jax.ShapeDtypeStruct((NCORE, npad, 16), jnp.float32),
        ],
        mesh=_mesh(),
        scratch_types=[
            pltpu.VMEM((GRP * CB, 128), jnp.int32),
            pltpu.VMEM((GRP * CB, 128), jnp.int32),
            pltpu.VMEM((CH, 16), jnp.float32),
            pltpu.VMEM((CH, 16), jnp.float32),
            pltpu.VMEM((CH // 8, 128), jnp.float32),
            pltpu.VMEM((16,), jnp.float32),
            pltpu.SemaphoreType.DMA,
            pltpu.VMEM_SHARED((npad, 16), jnp.float32),
        ],
        compiler_params=pltpu.CompilerParams(use_tc_tiling_on_sc=False),
    )
    return call(src2, dst2, as_t, ad_t, m16, zrows)


def _agg_phase(src2, dst2, e_t, h_t, inv_t, zrows, perm, ep, npad):
    cdim, hh, cph = OUT, 4, 16
    te = ep // NW
    nch = te // CH
    stripe = npad // NSUB

    def body(src_h, dst_h, e_h, h_h, inv_h, z_h, perm_h, acc_h,
             sidx, didx, hrow, erow, irow, permv, sem, acc_sh):
        c = lax.axis_index("c")
        s = lax.axis_index("s")
        wid = s * NCORE + c
        pltpu.sync_copy(z_h, acc_sh.at[pl.ds(s * stripe, stripe)])
        pltpu.sync_copy(perm_h, permv)
        plsc.subcore_barrier()
        pv = permv[...]

        def group(g, carry):
            rb = wid * (te // 128) + g * (GRP * CB)
            pltpu.sync_copy(src_h.at[pl.ds(rb, GRP * CB)], sidx)
            pltpu.sync_copy(dst_h.at[pl.ds(rb, GRP * CB)], didx)
            for sub in range(GRP):
                eb8 = wid * (te // 8) + (g * GRP + sub) * (CH // 8)
                descs = []
                for jj in range(CB):
                    j = sub * CB + jj
                    descs.append(pltpu.async_copy(
                        h_h.at[sidx.at[j]],
                        hrow.at[pl.ds(jj * 128, 128)], sem))
                    descs.append(pltpu.async_copy(
                        inv_h.at[didx.at[j]],
                        irow.at[pl.ds(jj * 128, 128)], sem))
                pltpu.sync_copy(e_h.at[pl.ds(eb8, CH // 8)], erow)
                for dsc in descs:
                    dsc.wait()

                def edge(k, carry2):
                    ev = erow[k // 8, pl.ds((k % 8) * 16, 16)]
                    av = lax.gather(
                        ev * irow[k, :], pv[:, None],
                        lax.GatherDimensionNumbers(
                            offset_dims=(), collapsed_slice_dims=(0,),
                            start_index_map=(0,)),
                        slice_sizes=(1,),
                        mode=lax.GatherScatterMode.PROMISE_IN_BOUNDS)
                    for h in range(hh):
                        a = av[h]
                        sl = pl.ds(h * cph, 16)
                        hrow[k, sl] = hrow[k, sl] * a
                    return carry2

                lax.fori_loop(0, CH, edge, 0)
                for jj in range(CB):
                    j = sub * CB + jj
                    pltpu.sync_copy(
                        hrow.at[pl.ds(jj * 128, 128)],
                        acc_sh.at[didx.at[j]], add=True)
            return carry

        lax.fori_loop(0, nch // GRP, group, 0)
        plsc.subcore_barrier()
        pltpu.sync_copy(acc_sh.at[pl.ds(s * stripe, stripe)],
                        acc_h.at[c, pl.ds(s * stripe, stripe)])

    call = pl.kernel(
        body,
        out_type=[
            jax.ShapeDtypeStruct((NCORE, npad, cdim), jnp.float32),
        ],
        mesh=_mesh(),
        scratch_types=[
            pltpu.VMEM((GRP * CB, 128), jnp.int32),
            pltpu.VMEM((GRP * CB, 128), jnp.int32),
            pltpu.VMEM((CH, cdim), jnp.float32),
            pltpu.VMEM((CH // 8, 128), jnp.float32),
            pltpu.VMEM((CH, 16), jnp.float32),
            pltpu.VMEM((16,), jnp.int32),
            pltpu.SemaphoreType.DMA,
            pltpu.VMEM_SHARED((npad, cdim), jnp.float32),
        ],
        compiler_params=pltpu.CompilerParams(use_tc_tiling_on_sc=False),
    )
    return call(src2, dst2, e_t, h_t, inv_t, zrows, perm)[0]


def _leaky(t):
    return jnp.where(t >= 0.0, t, 0.2 * t)


def kernel(x, edge_index, W1, a_src1, a_dst1, b1, W2, a_src2, a_dst2, b2):
    n = x.shape[0]
    e = edge_index.shape[1]
    total = e + n
    algn = CH * GRP
    te = -(-total // (NW * algn)) * algn
    ep = te * NW
    padc = ep - total
    np8 = -(-(n + 1) // (NSUB * 8)) * (NSUB * 8)
    npad = -(-(n + 8) // (NSUB * 8)) * (NSUB * 8)
    stripe = npad // NSUB

    ei = edge_index.astype(jnp.int32)
    loopv = jnp.arange(n, dtype=jnp.int32)
    src = jnp.concatenate([ei[0], loopv,
                           jnp.zeros((padc,), jnp.int32)])
    dst = jnp.concatenate([ei[1], loopv,
                           jnp.full((padc,), n, jnp.int32)])
    src2 = src.reshape(ep // 128, 128)
    dst2 = dst.reshape(ep // 128, 128)

    z16 = jnp.zeros((stripe, 16), jnp.float32)
    z64 = jnp.zeros((stripe, OUT), jnp.float32)
    p0 = jnp.arange(16, dtype=jnp.int32)
    p4 = jnp.minimum(p0 + 4, 15)

    ha, hb, s1, d1, m1 = _layer1_dense(x, W1, a_src1, a_dst1)
    hap = jnp.pad(ha, ((0, np8 - n), (0, 0)))
    hbp = jnp.pad(hb, ((0, np8 - n), (0, 0)))
    s1p = jnp.pad(s1, ((0, np8 - n), (0, 0)))
    d1p = jnp.pad(d1, ((0, np8 - n), (0, 0)))
    mval = _leaky(m1[0, 0] + m1[1, 0])
    m16 = jnp.broadcast_to(mval, (16,)).astype(jnp.float32)

    e1, den1 = _att_phase(src2, dst2, s1p, d1p, m16, z16, ep, npad)
    inv1 = _combine_inv(den1)
    acc1a = _agg_phase(src2, dst2, e1, hap, inv1, z64, p0, ep, npad)
    acc1b = _agg_phase(src2, dst2, e1, hbp, inv1, z64, p4, ep, npad)

    b1r = b1.reshape(1, 128)
    A2s = jnp.zeros((OUT, 16), jnp.float32).at[:, :4].set(a_src2[0][:, None])
    A2d = jnp.zeros((OUT, 16), jnp.float32).at[:, :4].set(a_dst2[0][:, None])
    h2, s2, d2, m2 = _layer2_dense(acc1a, acc1b, b1r, W2, A2s, A2d, n)
    h2p = jnp.pad(h2, ((0, np8 - n), (0, 0)))
    s2p = jnp.pad(s2, ((0, np8 - n), (0, 0)))
    d2p = jnp.pad(d2, ((0, np8 - n), (0, 0)))
    mval2 = _leaky(m2[0, 0] + m2[1, 0])
    m16b = jnp.broadcast_to(mval2, (16,)).astype(jnp.float32)

    e2, den2 = _att_phase(src2, dst2, s2p, d2p, m16b, z16, ep, npad)
    inv2 = _combine_inv(den2)
    acc2 = _agg_phase(src2, dst2, e2, h2p, inv2, z64, p0, ep, npad)

    return _final_sum(acc2, b2.reshape(1, OUT), n)


# speedup vs baseline: 24.2765x; 1.2969x over previous
import functools

import jax
import jax.numpy as jnp
from jax import lax
from jax.experimental import pallas as pl
from jax.experimental.pallas import tpu as pltpu
from jax.experimental.pallas import tpu_sc as plsc

H1 = 8
HID = 16
OUT = 64

NSUB = 16
NCORE = 2
NW = NSUB * NCORE
CH = 512
CB = CH // 128

BR = 1000

NEG = -3.0e38


def _tc1_body(x_ref, w_ref, asrc_ref, adst_ref, ha_ref, hb_ref,
              s_ref, d_ref, m_ref):
    i = pl.program_id(0)
    hb = jnp.dot(x_ref[...], w_ref[...], preferred_element_type=jnp.float32)
    ha_ref[...] = hb[:, :OUT]
    hb_ref[...] = hb[:, OUT:]
    h3 = hb.reshape(BR, H1, HID)
    sb = jnp.sum(h3 * asrc_ref[...][None], axis=-1)
    db = jnp.sum(h3 * adst_ref[...][None], axis=-1)
    z = jnp.zeros((BR, 16 - H1), jnp.float32)
    s_ref[...] = jnp.concatenate([sb, z], axis=1)
    d_ref[...] = jnp.concatenate([db, z], axis=1)
    mrow = jnp.concatenate(
        [jnp.full((1, 128), jnp.max(sb)), jnp.full((1, 128), jnp.max(db)),
         jnp.full((6, 128), NEG)], axis=0)

    @pl.when(i == 0)
    def _():
        m_ref[...] = mrow

    @pl.when(i > 0)
    def _():
        m_ref[...] = jnp.maximum(m_ref[...], mrow)


def _layer1_dense(x, W1, a_src1, a_dst1):
    n = x.shape[0]
    grid = n // BR
    return pl.pallas_call(
        _tc1_body,
        grid=(grid,),
        in_specs=[
            pl.BlockSpec((BR, 128), lambda i: (i, 0)),
            pl.BlockSpec((128, 128), lambda i: (0, 0)),
            pl.BlockSpec((H1, HID), lambda i: (0, 0)),
            pl.BlockSpec((H1, HID), lambda i: (0, 0)),
        ],
        out_specs=[
            pl.BlockSpec((BR, OUT), lambda i: (i, 0)),
            pl.BlockSpec((BR, OUT), lambda i: (i, 0)),
            pl.BlockSpec((BR, 16), lambda i: (i, 0)),
            pl.BlockSpec((BR, 16), lambda i: (i, 0)),
            pl.BlockSpec((8, 128), lambda i: (0, 0)),
        ],
        out_shape=[
            jax.ShapeDtypeStruct((n, OUT), jnp.float32),
            jax.ShapeDtypeStruct((n, OUT), jnp.float32),
            jax.ShapeDtypeStruct((n, 16), jnp.float32),
            jax.ShapeDtypeStruct((n, 16), jnp.float32),
            jax.ShapeDtypeStruct((8, 128), jnp.float32),
        ],
    )(x, W1, a_src1, a_dst1)


def _inv_body(d_ref, o_ref):
    o_ref[...] = 1.0 / (d_ref[0] + d_ref[1] + 1e-16)


def _combine_inv(den):
    npad = den.shape[1]
    return pl.pallas_call(
        _inv_body,
        out_shape=jax.ShapeDtypeStruct((npad, 16), jnp.float32),
    )(den)


def _tc2_body(a_ref, b_ref, b1_ref, w2_ref, a2s_ref, a2d_ref,
              h2_ref, s_ref, d_ref, m_ref):
    i = pl.program_id(0)
    o = jnp.concatenate([a_ref[0] + a_ref[1], b_ref[0] + b_ref[1]], axis=1)
    o = o + b1_ref[...]
    o = jnp.where(o > 0, o, jnp.exp(o) - 1.0)
    h2 = jnp.dot(o, w2_ref[...], preferred_element_type=jnp.float32)
    h2_ref[...] = h2
    sb = jnp.dot(h2, a2s_ref[...], preferred_element_type=jnp.float32)
    db = jnp.dot(h2, a2d_ref[...], preferred_element_type=jnp.float32)
    s_ref[...] = sb
    d_ref[...] = db
    mrow = jnp.concatenate(
        [jnp.full((1, 128), jnp.max(sb[:, 0])),
         jnp.full((1, 128), jnp.max(db[:, 0])),
         jnp.full((6, 128), NEG)], axis=0)

    @pl.when(i == 0)
    def _():
        m_ref[...] = mrow

    @pl.when(i > 0)
    def _():
        m_ref[...] = jnp.maximum(m_ref[...], mrow)


def _layer2_dense(acc1a, acc1b, b1r, W2, A2s, A2d, n):
    grid = n // BR
    return pl.pallas_call(
        _tc2_body,
        grid=(grid,),
        in_specs=[
            pl.BlockSpec((2, BR, OUT), lambda i: (0, i, 0)),
            pl.BlockSpec((2, BR, OUT), lambda i: (0, i, 0)),
            pl.BlockSpec((1, 128), lambda i: (0, 0)),
            pl.BlockSpec((128, OUT), lambda i: (0, 0)),
            pl.BlockSpec((OUT, 16), lambda i: (0, 0)),
            pl.BlockSpec((OUT, 16), lambda i: (0, 0)),
        ],
        out_specs=[
            pl.BlockSpec((BR, OUT), lambda i: (i, 0)),
            pl.BlockSpec((BR, 16), lambda i: (i, 0)),
            pl.BlockSpec((BR, 16), lambda i: (i, 0)),
            pl.BlockSpec((8, 128), lambda i: (0, 0)),
        ],
        out_shape=[
            jax.ShapeDtypeStruct((n, OUT), jnp.float32),
            jax.ShapeDtypeStruct((n, 16), jnp.float32),
            jax.ShapeDtypeStruct((n, 16), jnp.float32),
            jax.ShapeDtypeStruct((8, 128), jnp.float32),
        ],
    )(acc1a, acc1b, b1r, W2, A2s, A2d)


def _fin_body(a_ref, b_ref, o_ref):
    o_ref[...] = a_ref[0] + a_ref[1] + b_ref[...]


def _final_sum(acc2, b2r, n):
    grid = n // BR
    return pl.pallas_call(
        _fin_body,
        grid=(grid,),
        in_specs=[
            pl.BlockSpec((2, BR, OUT), lambda i: (0, i, 0)),
            pl.BlockSpec((1, OUT), lambda i: (0, 0)),
        ],
        out_specs=pl.BlockSpec((BR, OUT), lambda i: (i, 0)),
        out_shape=jax.ShapeDtypeStruct((n, OUT), jnp.float32),
    )(acc2, b2r)


def _mesh():
    return plsc.VectorSubcoreMesh(
        core_axis_name="c", subcore_axis_name="s",
        num_cores=NCORE, num_subcores=NSUB)


def _att_phase(cidx, as_t, ad_t, m16, zrows, ep, npad):
    te = ep // NW
    nch = te // CH
    stripe = npad // NSUB

    def body(ci_h, as_h, ad_h, m_h, z_h, e_h, den_h,
             sidx, srow, drow, erow, mv, sem, sem2, den_sh):
        c = lax.axis_index("c")
        s = lax.axis_index("s")
        wid = s * NCORE + c
        pltpu.sync_copy(z_h, den_sh.at[pl.ds(s * stripe, stripe)])
        pltpu.sync_copy(m_h, mv)
        plsc.subcore_barrier()
        m = mv[...]

        def chunk(i, carry):
            rb = (wid * nch + i) * (2 * CB)
            eb8 = wid * (te // 8) + i * (CH // 8)
            pltpu.sync_copy(ci_h.at[pl.ds(rb, 2 * CB)], sidx)
            descs = []
            for j in range(CB):
                descs.append(pltpu.async_copy(
                    as_h.at[sidx.at[j]],
                    srow.at[pl.ds(j * 128, 128)], sem))
                descs.append(pltpu.async_copy(
                    ad_h.at[sidx.at[CB + j]],
                    drow.at[pl.ds(j * 128, 128)], sem))
            for dsc in descs:
                dsc.wait()

            def edge(k, carry2):
                z = srow[k, :] + drow[k, :]
                zl = jnp.where(z >= 0.0, z, 0.2 * z) - m
                ev = jnp.exp(zl)
                srow[k, :] = ev
                erow[k // 8, pl.ds((k % 8) * 16, 16)] = ev
                return carry2

            lax.fori_loop(0, CH, edge, 0, unroll=8)

            pltpu.sync_copy(erow, e_h.at[pl.ds(eb8, CH // 8)])
            adds = []
            for j in range(CB):
                adds.append(pltpu.async_copy(
                    srow.at[pl.ds(j * 128, 128)],
                    den_sh.at[sidx.at[CB + j]], sem2, add=True))
            for dsc in adds:
                dsc.wait()
            return carry

        lax.fori_loop(0, nch, chunk, 0)
        plsc.subcore_barrier()
        pltpu.sync_copy(den_sh.at[pl.ds(s * stripe, stripe)],
                        den_h.at[c, pl.ds(s * stripe, stripe)])

    call = pl.kernel(
        body,
        out_type=[
            jax.ShapeDtypeStruct((ep // 8, 128), jnp.float32),
            jax.ShapeDtypeStruct((NCORE, npad, 16), jnp.float32),
        ],
        mesh=_mesh(),
        scratch_types=[
            pltpu.VMEM((2 * CB, 128), jnp.int32),
            pltpu.VMEM((CH, 16), jnp.float32),
            pltpu.VMEM((CH, 16), jnp.float32),
            pltpu.VMEM((CH // 8, 128), jnp.float32),
            pltpu.VMEM((16,), jnp.float32),
            pltpu.SemaphoreType.DMA,
            pltpu.SemaphoreType.DMA,
            pltpu.VMEM_SHARED((npad, 16), jnp.float32),
        ],
        compiler_params=pltpu.CompilerParams(use_tc_tiling_on_sc=False),
    )
    return call(cidx, as_t, ad_t, m16, zrows)


def _agg_phase(cidx, e_t, h_t, inv_t, zrows, perm, ep, npad):
    cdim, hh, cph = OUT, 4, 16
    te = ep // NW
    nch = te // CH
    stripe = npad // NSUB

    def body(ci_h, e_h, h_h, inv_h, z_h, perm_h, acc_h,
             sidx, hrow, erow, irow, permv, sem, sem2, acc_sh):
        c = lax.axis_index("c")
        s = lax.axis_index("s")
        wid = s * NCORE + c
        pltpu.sync_copy(z_h, acc_sh.at[pl.ds(s * stripe, stripe)])
        pltpu.sync_copy(perm_h, permv)
        plsc.subcore_barrier()
        pv = permv[...]

        def chunk(i, carry):
            rb = (wid * nch + i) * (2 * CB)
            eb8 = wid * (te // 8) + i * (CH // 8)
            pltpu.sync_copy(ci_h.at[pl.ds(rb, 2 * CB)], sidx)
            descs = []
            for j in range(CB):
                descs.append(pltpu.async_copy(
                    h_h.at[sidx.at[j]],
                    hrow.at[pl.ds(j * 128, 128)], sem))
                descs.append(pltpu.async_copy(
                    inv_h.at[sidx.at[CB + j]],
                    irow.at[pl.ds(j * 128, 128)], sem))
            pltpu.sync_copy(e_h.at[pl.ds(eb8, CH // 8)], erow)
            for dsc in descs:
                dsc.wait()

            def edge(k, carry2):
                ev = erow[k // 8, pl.ds((k % 8) * 16, 16)]
                av = lax.gather(
                    ev * irow[k, :], pv[:, None],
                    lax.GatherDimensionNumbers(
                        offset_dims=(), collapsed_slice_dims=(0,),
                        start_index_map=(0,)),
                    slice_sizes=(1,),
                    mode=lax.GatherScatterMode.PROMISE_IN_BOUNDS)
                for h in range(hh):
                    a = av[h]
                    sl = pl.ds(h * cph, 16)
                    hrow[k, sl] = hrow[k, sl] * a
                return carry2

            lax.fori_loop(0, CH, edge, 0, unroll=4)
            for j in range(CB):
                pltpu.sync_copy(
                    hrow.at[pl.ds(j * 128, 128)],
                    acc_sh.at[sidx.at[CB + j]], add=True)
            return carry

        lax.fori_loop(0, nch, chunk, 0)
        plsc.subcore_barrier()
        pltpu.sync_copy(acc_sh.at[pl.ds(s * stripe, stripe)],
                        acc_h.at[c, pl.ds(s * stripe, stripe)])

    call = pl.kernel(
        body,
        out_type=[
            jax.ShapeDtypeStruct((NCORE, npad, cdim), jnp.float32),
        ],
        mesh=_mesh(),
        scratch_types=[
            pltpu.VMEM((2 * CB, 128), jnp.int32),
            pltpu.VMEM((CH, cdim), jnp.float32),
            pltpu.VMEM((CH // 8, 128), jnp.float32),
            pltpu.VMEM((CH, 16), jnp.float32),
            pltpu.VMEM((16,), jnp.int32),
            pltpu.SemaphoreType.DMA,
            pltpu.SemaphoreType.DMA,
            pltpu.VMEM_SHARED((npad, cdim), jnp.float32),
        ],
        compiler_params=pltpu.CompilerParams(use_tc_tiling_on_sc=False),
    )
    return call(cidx, e_t, h_t, inv_t, zrows, perm)[0]


def _leaky(t):
    return jnp.where(t >= 0.0, t, 0.2 * t)


def kernel(x, edge_index, W1, a_src1, a_dst1, b1, W2, a_src2, a_dst2, b2):
    n = x.shape[0]
    e = edge_index.shape[1]
    total = e + n
    te = -(-total // (NW * CH)) * CH
    ep = te * NW
    padc = ep - total
    np8 = -(-(n + 1) // (NSUB * 8)) * (NSUB * 8)
    npad = -(-(n + 8) // (NSUB * 8)) * (NSUB * 8)
    stripe = npad // NSUB

    ei = edge_index.astype(jnp.int32)
    loopv = jnp.arange(n, dtype=jnp.int32)
    src = jnp.concatenate([ei[0], loopv,
                           jnp.zeros((padc,), jnp.int32)])
    dst = jnp.concatenate([ei[1], loopv,
                           jnp.full((padc,), n, jnp.int32)])
    nch = te // CH
    src3 = src.reshape(NW, nch, CB, 128)
    dst3 = dst.reshape(NW, nch, CB, 128)
    cidx = jnp.concatenate([src3, dst3], axis=2).reshape(-1, 128)

    z16 = jnp.zeros((stripe, 16), jnp.float32)
    z64 = jnp.zeros((stripe, OUT), jnp.float32)
    p0 = jnp.arange(16, dtype=jnp.int32)
    p4 = jnp.minimum(p0 + 4, 15)

    ha, hb, s1, d1, m1 = _layer1_dense(x, W1, a_src1, a_dst1)
    hap = jnp.pad(ha, ((0, np8 - n), (0, 0)))
    hbp = jnp.pad(hb, ((0, np8 - n), (0, 0)))
    s1p = jnp.pad(s1, ((0, np8 - n), (0, 0)))
    d1p = jnp.pad(d1, ((0, np8 - n), (0, 0)))
    mval = _leaky(m1[0, 0] + m1[1, 0])
    m16 = jnp.broadcast_to(mval, (16,)).astype(jnp.float32)

    e1, den1 = _att_phase(cidx, s1p, d1p, m16, z16, ep, npad)
    inv1 = _combine_inv(den1)
    acc1a = _agg_phase(cidx, e1, hap, inv1, z64, p0, ep, npad)
    z64b = z64 + acc1a[0, :1, :1] * 0.0
    acc1b = _agg_phase(cidx, e1, hbp, inv1, z64b, p4, ep, npad)

    b1r = b1.reshape(1, 128)
    A2s = jnp.zeros((OUT, 16), jnp.float32).at[:, :4].set(a_src2[0][:, None])
    A2d = jnp.zeros((OUT, 16), jnp.float32).at[:, :4].set(a_dst2[0][:, None])
    h2, s2, d2, m2 = _layer2_dense(acc1a, acc1b, b1r, W2, A2s, A2d, n)
    h2p = jnp.pad(h2, ((0, np8 - n), (0, 0)))
    s2p = jnp.pad(s2, ((0, np8 - n), (0, 0)))
    d2p = jnp.pad(d2, ((0, np8 - n), (0, 0)))
    mval2 = _leaky(m2[0, 0] + m2[1, 0])
    m16b = jnp.broadcast_to(mval2, (16,)).astype(jnp.float32)

    e2, den2 = _att_phase(cidx, s2p, d2p, m16b, z16, ep, npad)
    inv2 = _combine_inv(den2)
    acc2 = _agg_phase(cidx, e2, h2p, inv2, z64, p0, ep, npad)

    return _final_sum(acc2, b2.reshape(1, OUT), n)


# speedup vs baseline: 24.3296x; 1.0022x over previous
import functools

import jax
import jax.numpy as jnp
from jax import lax
from jax.experimental import pallas as pl
from jax.experimental.pallas import tpu as pltpu
from jax.experimental.pallas import tpu_sc as plsc

H1 = 8
HID = 16
OUT = 64

NSUB = 16
NCORE = 2
NW = NSUB * NCORE
CH = 512
CB = CH // 128

BR = 1000

NEG = -3.0e38


def _tc1_body(x_ref, w_ref, asrc_ref, adst_ref, ha_ref, hb_ref,
              s_ref, d_ref, m_ref):
    i = pl.program_id(0)
    hb = jnp.dot(x_ref[...], w_ref[...], preferred_element_type=jnp.float32)
    ha_ref[...] = hb[:, :OUT]
    hb_ref[...] = hb[:, OUT:]
    h3 = hb.reshape(BR, H1, HID)
    sb = jnp.sum(h3 * asrc_ref[...][None], axis=-1)
    db = jnp.sum(h3 * adst_ref[...][None], axis=-1)
    z = jnp.zeros((BR, 16 - H1), jnp.float32)
    s_ref[...] = jnp.concatenate([sb, z], axis=1)
    d_ref[...] = jnp.concatenate([db, z], axis=1)
    mrow = jnp.concatenate(
        [jnp.full((1, 128), jnp.max(sb)), jnp.full((1, 128), jnp.max(db)),
         jnp.full((6, 128), NEG)], axis=0)

    @pl.when(i == 0)
    def _():
        m_ref[...] = mrow

    @pl.when(i > 0)
    def _():
        m_ref[...] = jnp.maximum(m_ref[...], mrow)


def _layer1_dense(x, W1, a_src1, a_dst1):
    n = x.shape[0]
    grid = n // BR
    return pl.pallas_call(
        _tc1_body,
        grid=(grid,),
        in_specs=[
            pl.BlockSpec((BR, 128), lambda i: (i, 0)),
            pl.BlockSpec((128, 128), lambda i: (0, 0)),
            pl.BlockSpec((H1, HID), lambda i: (0, 0)),
            pl.BlockSpec((H1, HID), lambda i: (0, 0)),
        ],
        out_specs=[
            pl.BlockSpec((BR, OUT), lambda i: (i, 0)),
            pl.BlockSpec((BR, OUT), lambda i: (i, 0)),
            pl.BlockSpec((BR, 16), lambda i: (i, 0)),
            pl.BlockSpec((BR, 16), lambda i: (i, 0)),
            pl.BlockSpec((8, 128), lambda i: (0, 0)),
        ],
        out_shape=[
            jax.ShapeDtypeStruct((n, OUT), jnp.float32),
            jax.ShapeDtypeStruct((n, OUT), jnp.float32),
            jax.ShapeDtypeStruct((n, 16), jnp.float32),
            jax.ShapeDtypeStruct((n, 16), jnp.float32),
            jax.ShapeDtypeStruct((8, 128), jnp.float32),
        ],
    )(x, W1, a_src1, a_dst1)


def _inv_body(d_ref, o_ref):
    o_ref[...] = 1.0 / (d_ref[0] + d_ref[1] + 1e-16)


def _combine_inv(den):
    npad = den.shape[1]
    return pl.pallas_call(
        _inv_body,
        out_shape=jax.ShapeDtypeStruct((npad, 16), jnp.float32),
    )(den)


def _tc2_body(a_ref, b_ref, b1_ref, w2_ref, a2s_ref, a2d_ref,
              h2_ref, s_ref, d_ref, m_ref):
    i = pl.program_id(0)
    o = jnp.concatenate([a_ref[0] + a_ref[1], b_ref[0] + b_ref[1]], axis=1)
    o = o + b1_ref[...]
    o = jnp.where(o > 0, o, jnp.exp(o) - 1.0)
    h2 = jnp.dot(o, w2_ref[...], preferred_element_type=jnp.float32)
    h2_ref[...] = h2
    sb = jnp.dot(h2, a2s_ref[...], preferred_element_type=jnp.float32)
    db = jnp.dot(h2, a2d_ref[...], preferred_element_type=jnp.float32)
    s_ref[...] = sb
    d_ref[...] = db
    mrow = jnp.concatenate(
        [jnp.full((1, 128), jnp.max(sb[:, 0])),
         jnp.full((1, 128), jnp.max(db[:, 0])),
         jnp.full((6, 128), NEG)], axis=0)

    @pl.when(i == 0)
    def _():
        m_ref[...] = mrow

    @pl.when(i > 0)
    def _():
        m_ref[...] = jnp.maximum(m_ref[...], mrow)


def _layer2_dense(acc1a, acc1b, b1r, W2, A2s, A2d, n):
    grid = n // BR
    return pl.pallas_call(
        _tc2_body,
        grid=(grid,),
        in_specs=[
            pl.BlockSpec((2, BR, OUT), lambda i: (0, i, 0)),
            pl.BlockSpec((2, BR, OUT), lambda i: (0, i, 0)),
            pl.BlockSpec((1, 128), lambda i: (0, 0)),
            pl.BlockSpec((128, OUT), lambda i: (0, 0)),
            pl.BlockSpec((OUT, 16), lambda i: (0, 0)),
            pl.BlockSpec((OUT, 16), lambda i: (0, 0)),
        ],
        out_specs=[
            pl.BlockSpec((BR, OUT), lambda i: (i, 0)),
            pl.BlockSpec((BR, 16), lambda i: (i, 0)),
            pl.BlockSpec((BR, 16), lambda i: (i, 0)),
            pl.BlockSpec((8, 128), lambda i: (0, 0)),
        ],
        out_shape=[
            jax.ShapeDtypeStruct((n, OUT), jnp.float32),
            jax.ShapeDtypeStruct((n, 16), jnp.float32),
            jax.ShapeDtypeStruct((n, 16), jnp.float32),
            jax.ShapeDtypeStruct((8, 128), jnp.float32),
        ],
    )(acc1a, acc1b, b1r, W2, A2s, A2d)


def _fin_body(a_ref, b_ref, o_ref):
    o_ref[...] = a_ref[0] + a_ref[1] + b_ref[...]


def _final_sum(acc2, b2r, n):
    grid = n // BR
    return pl.pallas_call(
        _fin_body,
        grid=(grid,),
        in_specs=[
            pl.BlockSpec((2, BR, OUT), lambda i: (0, i, 0)),
            pl.BlockSpec((1, OUT), lambda i: (0, 0)),
        ],
        out_specs=pl.BlockSpec((BR, OUT), lambda i: (i, 0)),
        out_shape=jax.ShapeDtypeStruct((n, OUT), jnp.float32),
    )(acc2, b2r)


def _mesh():
    return plsc.VectorSubcoreMesh(
        core_axis_name="c", subcore_axis_name="s",
        num_cores=NCORE, num_subcores=NSUB)


def _att_phase(cidx, as_t, ad_t, m16, zrows, ep, npad):
    te = ep // NW
    nch = te // CH
    stripe = npad // NSUB

    def body(ci_h, as_h, ad_h, m_h, z_h, e_h, den_h,
             sidx, srow, drow, erow, mv, sem, sem2, den_sh):
        c = lax.axis_index("c")
        s = lax.axis_index("s")
        wid = s * NCORE + c
        pltpu.sync_copy(z_h, den_sh.at[pl.ds(s * stripe, stripe)])
        pltpu.sync_copy(m_h, mv)
        plsc.subcore_barrier()
        m = mv[...]

        def chunk(i, carry):
            rb = (wid * nch + i) * (2 * CB)
            eb8 = wid * (te // 8) + i * (CH // 8)
            pltpu.sync_copy(ci_h.at[pl.ds(rb, 2 * CB)], sidx)
            descs = []
            for j in range(CB):
                descs.append(pltpu.async_copy(
                    as_h.at[sidx.at[j]],
                    srow.at[pl.ds(j * 128, 128)], sem))
                descs.append(pltpu.async_copy(
                    ad_h.at[sidx.at[CB + j]],
                    drow.at[pl.ds(j * 128, 128)], sem))
            for dsc in descs:
                dsc.wait()

            def edge(k, carry2):
                z = srow[k, :] + drow[k, :]
                zl = jnp.where(z >= 0.0, z, 0.2 * z) - m
                ev = jnp.exp(zl)
                srow[k, :] = ev
                erow[k // 8, pl.ds((k % 8) * 16, 16)] = ev
                return carry2

            lax.fori_loop(0, CH, edge, 0, unroll=8)

            pltpu.sync_copy(erow, e_h.at[pl.ds(eb8, CH // 8)])
            adds = []
            for j in range(CB):
                adds.append(pltpu.async_copy(
                    srow.at[pl.ds(j * 128, 128)],
                    den_sh.at[sidx.at[CB + j]], sem2, add=True))
            for dsc in adds:
                dsc.wait()
            return carry

        lax.fori_loop(0, nch, chunk, 0)
        plsc.subcore_barrier()
        pltpu.sync_copy(den_sh.at[pl.ds(s * stripe, stripe)],
                        den_h.at[c, pl.ds(s * stripe, stripe)])

    call = pl.kernel(
        body,
        out_type=[
            jax.ShapeDtypeStruct((ep // 8, 128), jnp.float32),
            jax.ShapeDtypeStruct((NCORE, npad, 16), jnp.float32),
        ],
        mesh=_mesh(),
        scratch_types=[
            pltpu.VMEM((2 * CB, 128), jnp.int32),
            pltpu.VMEM((CH, 16), jnp.float32),
            pltpu.VMEM((CH, 16), jnp.float32),
            pltpu.VMEM((CH // 8, 128), jnp.float32),
            pltpu.VMEM((16,), jnp.float32),
            pltpu.SemaphoreType.DMA,
            pltpu.SemaphoreType.DMA,
            pltpu.VMEM_SHARED((npad, 16), jnp.float32),
        ],
        compiler_params=pltpu.CompilerParams(use_tc_tiling_on_sc=False),
    )
    return call(cidx, as_t, ad_t, m16, zrows)


def _agg_phase(cidx, e_t, h_t, inv_t, zrows, perm, ep, npad):
    cdim, hh, cph = OUT, 4, 16
    te = ep // NW
    nch = te // CH
    stripe = npad // NSUB

    def body(ci_h, e_h, h_h, inv_h, z_h, perm_h, acc_h,
             sidx, hrow, erow, irow, permv, sem, sem2, acc_sh):
        c = lax.axis_index("c")
        s = lax.axis_index("s")
        wid = s * NCORE + c
        pltpu.sync_copy(z_h, acc_sh.at[pl.ds(s * stripe, stripe)])
        pltpu.sync_copy(perm_h, permv)
        plsc.subcore_barrier()
        pv = permv[...]

        def chunk(i, carry):
            rb = (wid * nch + i) * (2 * CB)
            eb8 = wid * (te // 8) + i * (CH // 8)
            pltpu.sync_copy(ci_h.at[pl.ds(rb, 2 * CB)], sidx)
            descs = []
            for j in range(CB):
                descs.append(pltpu.async_copy(
                    h_h.at[sidx.at[j]],
                    hrow.at[pl.ds(j * 128, 128)], sem))
                descs.append(pltpu.async_copy(
                    inv_h.at[sidx.at[CB + j]],
                    irow.at[pl.ds(j * 128, 128)], sem))
            pltpu.sync_copy(e_h.at[pl.ds(eb8, CH // 8)], erow)
            for dsc in descs:
                dsc.wait()

            def edge(k, carry2):
                ev = erow[k // 8, pl.ds((k % 8) * 16, 16)]
                av = lax.gather(
                    ev * irow[k, :], pv[:, None],
                    lax.GatherDimensionNumbers(
                        offset_dims=(), collapsed_slice_dims=(0,),
                        start_index_map=(0,)),
                    slice_sizes=(1,),
                    mode=lax.GatherScatterMode.PROMISE_IN_BOUNDS)
                for h in range(hh):
                    a = av[h]
                    sl = pl.ds(h * cph, 16)
                    hrow[k, sl] = hrow[k, sl] * a
                return carry2

            lax.fori_loop(0, CH, edge, 0, unroll=8)
            for j in range(CB):
                pltpu.sync_copy(
                    hrow.at[pl.ds(j * 128, 128)],
                    acc_sh.at[sidx.at[CB + j]], add=True)
            return carry

        lax.fori_loop(0, nch, chunk, 0)
        plsc.subcore_barrier()
        pltpu.sync_copy(acc_sh.at[pl.ds(s * stripe, stripe)],
                        acc_h.at[c, pl.ds(s * stripe, stripe)])

    call = pl.kernel(
        body,
        out_type=[
            jax.ShapeDtypeStruct((NCORE, npad, cdim), jnp.float32),
        ],
        mesh=_mesh(),
        scratch_types=[
            pltpu.VMEM((2 * CB, 128), jnp.int32),
            pltpu.VMEM((CH, cdim), jnp.float32),
            pltpu.VMEM((CH // 8, 128), jnp.float32),
            pltpu.VMEM((CH, 16), jnp.float32),
            pltpu.VMEM((16,), jnp.int32),
            pltpu.SemaphoreType.DMA,
            pltpu.SemaphoreType.DMA,
            pltpu.VMEM_SHARED((npad, cdim), jnp.float32),
        ],
        compiler_params=pltpu.CompilerParams(use_tc_tiling_on_sc=False),
    )
    return call(cidx, e_t, h_t, inv_t, zrows, perm)[0]


def _leaky(t):
    return jnp.where(t >= 0.0, t, 0.2 * t)


def kernel(x, edge_index, W1, a_src1, a_dst1, b1, W2, a_src2, a_dst2, b2):
    n = x.shape[0]
    e = edge_index.shape[1]
    total = e + n
    te = -(-total // (NW * CH)) * CH
    ep = te * NW
    padc = ep - total
    np8 = -(-(n + 1) // (NSUB * 8)) * (NSUB * 8)
    npad = -(-(n + 8) // (NSUB * 8)) * (NSUB * 8)
    stripe = npad // NSUB

    ei = edge_index.astype(jnp.int32)
    loopv = jnp.arange(n, dtype=jnp.int32)
    src = jnp.concatenate([ei[0], loopv,
                           jnp.zeros((padc,), jnp.int32)])
    dst = jnp.concatenate([ei[1], loopv,
                           jnp.full((padc,), n, jnp.int32)])
    nch = te // CH
    src3 = src.reshape(NW, nch, CB, 128)
    dst3 = dst.reshape(NW, nch, CB, 128)
    cidx = jnp.concatenate([src3, dst3], axis=2).reshape(-1, 128)

    z16 = jnp.zeros((stripe, 16), jnp.float32)
    z64 = jnp.zeros((stripe, OUT), jnp.float32)
    p0 = jnp.arange(16, dtype=jnp.int32)
    p4 = jnp.minimum(p0 + 4, 15)

    ha, hb, s1, d1, m1 = _layer1_dense(x, W1, a_src1, a_dst1)
    hap = jnp.pad(ha, ((0, np8 - n), (0, 0)))
    hbp = jnp.pad(hb, ((0, np8 - n), (0, 0)))
    s1p = jnp.pad(s1, ((0, np8 - n), (0, 0)))
    d1p = jnp.pad(d1, ((0, np8 - n), (0, 0)))
    mval = _leaky(m1[0, 0] + m1[1, 0])
    m16 = jnp.broadcast_to(mval, (16,)).astype(jnp.float32)

    e1, den1 = _att_phase(cidx, s1p, d1p, m16, z16, ep, npad)
    inv1 = _combine_inv(den1)
    acc1a = _agg_phase(cidx, e1, hap, inv1, z64, p0, ep, npad)
    z64b = z64 + acc1a[0, :1, :1] * 0.0
    acc1b = _agg_phase(cidx, e1, hbp, inv1, z64b, p4, ep, npad)

    b1r = b1.reshape(1, 128)
    A2s = jnp.zeros((OUT, 16), jnp.float32).at[:, :4].set(a_src2[0][:, None])
    A2d = jnp.zeros((OUT, 16), jnp.float32).at[:, :4].set(a_dst2[0][:, None])
    h2, s2, d2, m2 = _layer2_dense(acc1a, acc1b, b1r, W2, A2s, A2d, n)
    h2p = jnp.pad(h2, ((0, np8 - n), (0, 0)))
    s2p = jnp.pad(s2, ((0, np8 - n), (0, 0)))
    d2p = jnp.pad(d2, ((0, np8 - n), (0, 0)))
    mval2 = _leaky(m2[0, 0] + m2[1, 0])
    m16b = jnp.broadcast_to(mval2, (16,)).astype(jnp.float32)

    e2, den2 = _att_phase(cidx, s2p, d2p, m16b, z16, ep, npad)
    inv2 = _combine_inv(den2)
    acc2 = _agg_phase(cidx, e2, h2p, inv2, z64, p0, ep, npad)

    return _final_sum(acc2, b2.reshape(1, OUT), n)
